# Initial kernel scaffold; baseline (speedup 1.0000x reference)
import jax
import jax.numpy as jnp
from jax.experimental import pallas as pl


def kernel(sentence_embs, paragraph_embs, document_embs, sent_valid_mask, para_valid_mask, Wp, bp, pos_emb, W1, b1, W2, b2):
    raise NotImplementedError("write your pallas kernel here")


# speedup vs baseline: 1.1145x; 1.1145x over previous
import functools

import jax
import jax.numpy as jnp
from jax import lax
from jax.experimental import pallas as pl
from jax.experimental.pallas import tpu as pltpu
from jax.experimental.pallas import tpu_sc as plsc

_H = 768
_TOPK = 50
_KPAD = 56
_MIN_SIG = 0.05
_B, _N, _P, _S = 4, 8, 8, 16
_NPS = _N * _P * _S
_HH = _H // 2

_PREC = lax.Precision.DEFAULT

_NC, _NS = 2, 16
_NW = _NC * _NS
_ROWS_PER_W = 8
_PAD_ROWS = _NW * _ROWS_PER_W


def _bf(x):
    return x.astype(jnp.bfloat16).astype(jnp.float32)


def _dot(a, b):
    return lax.dot_general(a.astype(jnp.bfloat16), b.astype(jnp.bfloat16),
                           (((1,), (0,)), ((), ())), precision=_PREC,
                           preferred_element_type=jnp.float32)


def _score_body(emb_ref, doc_ref, pos_ref, wp_ref, bp_ref, w1a_ref, w1b_ref,
                w1c_ref, b1_ref, w2_ref, b2_ref, out_ref):
    emb = emb_ref[...]
    doc = doc_ref[0]
    pos = pos_ref[...]

    pooled = (jnp.sum(doc, axis=0, keepdims=True) - doc) * (1.0 / (_N - 1))
    base = jnp.tanh(_dot(pooled, wp_ref[...]) + bp_ref[...])
    pred = (base.reshape(_N, 1, _H) + pos.reshape(1, _S, _H)).reshape(
        _N * _S, _H)
    pnorm = jnp.clip(jnp.sqrt(jnp.sum(pred * pred, axis=1, keepdims=True)),
                     1e-12, None)
    pn = pred / pnorm
    pn_w1 = _dot(pn, w1b_ref[...])

    anorm = jnp.clip(jnp.sqrt(jnp.sum(emb * emb, axis=1, keepdims=True)),
                     1e-12, None)
    an = emb / anorm
    a_w1 = _dot(an, w1a_ref[...])

    pnb = jnp.broadcast_to(pn.reshape(_N, 1, _S, _H),
                           (_N, _P, _S, _H)).reshape(_NPS, _H)
    cos = jnp.sum(an * pnb, axis=1, keepdims=True)
    cdis = jnp.clip(1.0 - cos, 0.0, 2.0) * 0.5

    pn_w1b = jnp.broadcast_to(pn_w1.reshape(_N, 1, _S, _HH),
                              (_N, _P, _S, _HH)).reshape(_NPS, _HH)
    h_pre = a_w1 + pn_w1b + _bf(cdis) * _bf(w1c_ref[...]) + b1_ref[...]
    h = 0.5 * h_pre * (1.0 + lax.erf(h_pre * (2.0 ** -0.5)))
    d0 = _dot(h, w2_ref[...])[:, 0:1]
    refined = jax.nn.sigmoid(d0 + b2_ref[...])
    score = 0.5 * cdis + 0.5 * refined
    out_ref[...] = jnp.concatenate(
        [score, cos, cdis, a_w1[:, 0:1], pn_w1b[:, 0:1], h_pre[:, 0:1],
         h[:, 0:1], d0], axis=1).reshape(1, _NPS, 8)


def _scores_call(emb_flat, doc, pos, wp, bp, w1a, w1b, w1c, b1, w2p, b2):
    spec_w = lambda shape: pl.BlockSpec(shape, lambda b: (0,) * len(shape))
    return pl.pallas_call(
        _score_body,
        grid=(_B,),
        in_specs=[
            pl.BlockSpec((_NPS, _H), lambda b: (b, 0)),
            pl.BlockSpec((1, _N, _H), lambda b: (b, 0, 0)),
            spec_w((_S, _H)),
            spec_w((_H, _H)),
            spec_w((1, _H)),
            spec_w((_H, _HH)),
            spec_w((_H, _HH)),
            spec_w((1, _HH)),
            spec_w((1, _HH)),
            spec_w((_HH, 128)),
            spec_w((1, 1)),
        ],
        out_specs=pl.BlockSpec((1, _NPS, 8), lambda b: (b, 0, 0)),
        out_shape=jax.ShapeDtypeStruct((_B, _NPS, 8), jnp.float32),
    )(emb_flat, doc, pos, wp, bp, w1a, w1b, w1c, b1, w2p, b2)


def _topk_body(srow_ref, scol_ref, sal_ref, psal_ref, sc_ref, w_ref, idx_ref,
               salc_ref):
    b = pl.program_id(0)
    srow = srow_ref[0]
    scol = scol_ref[0]

    smin = jnp.min(srow)
    smax = jnp.max(srow)
    spread = smax - smin
    is_norm = spread > _MIN_SIG
    denom = 1.0 / jnp.clip(spread, 1e-9, None)
    sal_row = jnp.where(is_norm, (srow - smin) * denom,
                        jnp.full_like(srow, 1.0 / _NPS))
    sal_col = jnp.where(is_norm, (scol - smin) * denom,
                        jnp.full_like(scol, 1.0 / _NPS))
    sal_ref[...] = sal_row.reshape(1, 1, _NPS)
    salc_ref[...] = sal_col

    psal = jnp.maximum(
        jnp.max(sal_col.reshape(_N * _P, _S, 1), axis=1), 0.0)
    psal_ref[...] = psal.reshape(1, _N * _P, 1)

    jj_row = lax.broadcasted_iota(jnp.int32, (8, _NPS), 1)
    ii_base = lax.broadcasted_iota(jnp.int32, (8, _NPS), 0)

    def _count_body(c, colsum):
        s8 = salc_ref[pl.ds(c * 8, 8), :]
        ii8 = c * 8 + ii_base
        beats = (sal_row > s8) | ((sal_row == s8) & (jj_row < ii8))
        return colsum + jnp.sum(beats.astype(jnp.float32), axis=0,
                                keepdims=True)

    colsum = lax.fori_loop(0, _NPS // 8, _count_body,
                           jnp.zeros((1, _NPS), jnp.float32))
    rank_row = (_NPS - 1.0) - colsum

    jj_f = jj_row.astype(jnp.float32)
    sc_parts = []
    ix_parts = []
    for c in range(_KPAD // 8):
        kk8 = (c * 8 + ii_base).astype(jnp.float32)
        oh = rank_row == kk8
        sc_parts.append(jnp.sum(jnp.where(oh, sal_row, 0.0), axis=1,
                                keepdims=True))
        ix_parts.append(jnp.sum(jnp.where(oh, jj_f, 0.0), axis=1,
                                keepdims=True))
    topk_sc = jnp.concatenate(sc_parts, axis=0)
    topk_if = jnp.concatenate(ix_parts, axis=0)
    sc_ref[...] = topk_sc.reshape(1, _KPAD, 1)
    idx_ref[...] = (topk_if.astype(jnp.int32) + b * _NPS).reshape(1, _KPAD, 1)

    kmask = lax.broadcasted_iota(jnp.int32, (_KPAD, 1), 0) < _TOPK
    wmax = jnp.max(jnp.where(kmask, topk_sc, -jnp.inf), axis=0, keepdims=True)
    wexp = jnp.where(kmask, jnp.exp(topk_sc - wmax), 0.0)
    w = wexp / jnp.sum(wexp, axis=0, keepdims=True)
    w_ref[...] = w.reshape(1, _KPAD, 1)


def _topk_call(scores):
    s_row = scores.reshape(_B, 1, _NPS)
    return pl.pallas_call(
        _topk_body,
        grid=(_B,),
        in_specs=[
            pl.BlockSpec((1, 1, _NPS), lambda b: (b, 0, 0)),
            pl.BlockSpec((1, _NPS, 1), lambda b: (b, 0, 0)),
        ],
        out_specs=[
            pl.BlockSpec((1, 1, _NPS), lambda b: (b, 0, 0)),
            pl.BlockSpec((1, _N * _P, 1), lambda b: (b, 0, 0)),
            pl.BlockSpec((1, _KPAD, 1), lambda b: (b, 0, 0)),
            pl.BlockSpec((1, _KPAD, 1), lambda b: (b, 0, 0)),
            pl.BlockSpec((1, _KPAD, 1), lambda b: (b, 0, 0)),
        ],
        out_shape=[
            jax.ShapeDtypeStruct((_B, 1, _NPS), jnp.float32),
            jax.ShapeDtypeStruct((_B, _N * _P, 1), jnp.float32),
            jax.ShapeDtypeStruct((_B, _KPAD, 1), jnp.float32),
            jax.ShapeDtypeStruct((_B, _KPAD, 1), jnp.float32),
            jax.ShapeDtypeStruct((_B, _KPAD, 1), jnp.int32),
        ],
        scratch_shapes=[pltpu.VMEM((_NPS, 1), jnp.float32)],
    )(s_row, scores)


def _gather_topk(table, idx_pad):
    mesh = plsc.VectorSubcoreMesh(core_axis_name="c", subcore_axis_name="s")

    @functools.partial(
        pl.kernel, mesh=mesh,
        out_type=jax.ShapeDtypeStruct((_PAD_ROWS, _H), jnp.float32),
        scratch_types=[
            pltpu.VMEM((_ROWS_PER_W,), jnp.int32),
            pltpu.VMEM((_ROWS_PER_W, _H), jnp.float32),
            pltpu.SemaphoreType.DMA,
        ],
    )
    def k(table_hbm, idx_hbm, out_hbm, idx_v, rows_v, sem):
        wid = lax.axis_index("s") * _NC + lax.axis_index("c")
        base = wid * _ROWS_PER_W
        pltpu.sync_copy(idx_hbm.at[pl.ds(base, _ROWS_PER_W)], idx_v)
        pltpu.async_copy(table_hbm.at[idx_v], rows_v, sem).wait()
        pltpu.sync_copy(rows_v, out_hbm.at[pl.ds(base, _ROWS_PER_W)])

    return k(table, idx_pad)


def kernel(sentence_embs, paragraph_embs, document_embs, sent_valid_mask,
           para_valid_mask, Wp, bp, pos_emb, W1, b1, W2, b2):
    emb_flat = sentence_embs.reshape(_B * _NPS, _H)
    w1a = W1[:_H, :]
    w1b = W1[_H:2 * _H, :]
    w1c = W1[2 * _H:, :]
    w2p = jnp.zeros((_HH, 128), jnp.float32).at[:, 0:1].set(W2)

    scores8 = _scores_call(emb_flat, document_embs, pos_emb, Wp,
                           bp.reshape(1, _H), w1a, w1b, w1c,
                           b1.reshape(1, _HH), w2p, b2.reshape(1, 1))
    scores = scores8[:, :, 0:1]

    sal, psal, topk_sc, topk_w, topk_idx = _topk_call(scores)

    idx_flat = topk_idx.reshape(_B, _KPAD)[:, :_TOPK].reshape(_B * _TOPK)
    idx_pad = jnp.zeros((_PAD_ROWS,), jnp.int32).at[:_B * _TOPK].set(idx_flat)
    gathered = _gather_topk(emb_flat, idx_pad)

    return (sal.reshape(_B, _N, _P, _S),
            psal.reshape(_B, _N, _P),
            gathered[:_B * _TOPK].reshape(_B, _TOPK, _H),
            topk_w.reshape(_B, _KPAD)[:, :_TOPK],
            topk_sc.reshape(_B, _KPAD)[:, :_TOPK])


# speedup vs baseline: 1.5769x; 1.4148x over previous
import functools

import jax
import jax.numpy as jnp
from jax import lax
from jax.experimental import pallas as pl
from jax.experimental.pallas import tpu as pltpu
from jax.experimental.pallas import tpu_sc as plsc

_H = 768
_TOPK = 50
_KPAD = 56
_MIN_SIG = 0.05
_B, _N, _P, _S = 4, 8, 8, 16
_NPS = _N * _P * _S
_HH = _H // 2

_PREC = lax.Precision.DEFAULT

_NC, _NS = 2, 16
_NW = _NC * _NS
_ROWS_PER_W = 8
_PAD_ROWS = _NW * _ROWS_PER_W


def _bf(x):
    return x.astype(jnp.bfloat16).astype(jnp.float32)


def _dot(a, b):
    return lax.dot_general(a.astype(jnp.bfloat16), b.astype(jnp.bfloat16),
                           (((1,), (0,)), ((), ())), precision=_PREC,
                           preferred_element_type=jnp.float32)


def _score_body(emb_ref, doc_ref, pos_ref, wp_ref, bp_ref, w1a_ref, w1b_ref,
                w1c_ref, b1_ref, w2_ref, b2_ref, out_ref):
    emb = emb_ref[...]
    doc = doc_ref[0]
    pos = pos_ref[...]

    pooled = (jnp.sum(doc, axis=0, keepdims=True) - doc) * (1.0 / (_N - 1))
    base = jnp.tanh(_dot(pooled, wp_ref[...]) + bp_ref[...])
    pred = (base.reshape(_N, 1, _H) + pos.reshape(1, _S, _H)).reshape(
        _N * _S, _H)
    pnorm = jnp.clip(jnp.sqrt(jnp.sum(pred * pred, axis=1, keepdims=True)),
                     1e-12, None)
    pn = pred / pnorm
    pn_w1 = _dot(pn, w1b_ref[...])

    anorm = jnp.clip(jnp.sqrt(jnp.sum(emb * emb, axis=1, keepdims=True)),
                     1e-12, None)
    an = emb / anorm
    a_w1 = _dot(an, w1a_ref[...])

    pnb = jnp.broadcast_to(pn.reshape(_N, 1, _S, _H),
                           (_N, _P, _S, _H)).reshape(_NPS, _H)
    cos = jnp.sum(an * pnb, axis=1, keepdims=True)
    cdis = jnp.clip(1.0 - cos, 0.0, 2.0) * 0.5

    pn_w1b = jnp.broadcast_to(pn_w1.reshape(_N, 1, _S, _HH),
                              (_N, _P, _S, _HH)).reshape(_NPS, _HH)
    h_pre = a_w1 + pn_w1b + _bf(cdis) * _bf(w1c_ref[...]) + b1_ref[...]
    h = 0.5 * h_pre * (1.0 + lax.erf(h_pre * (2.0 ** -0.5)))
    d0 = _dot(h, w2_ref[...])[:, 0:1]
    refined = jax.nn.sigmoid(d0 + b2_ref[...])
    score = 0.5 * cdis + 0.5 * refined
    out_ref[...] = jnp.concatenate(
        [score, cos, cdis, a_w1[:, 0:1], pn_w1b[:, 0:1], h_pre[:, 0:1],
         h[:, 0:1], d0], axis=1).reshape(1, _NPS, 8)


def _scores_call(emb_flat, doc, pos, wp, bp, w1a, w1b, w1c, b1, w2p, b2):
    spec_w = lambda shape: pl.BlockSpec(shape, lambda b: (0,) * len(shape))
    return pl.pallas_call(
        _score_body,
        grid=(_B,),
        in_specs=[
            pl.BlockSpec((_NPS, _H), lambda b: (b, 0)),
            pl.BlockSpec((1, _N, _H), lambda b: (b, 0, 0)),
            spec_w((_S, _H)),
            spec_w((_H, _H)),
            spec_w((1, _H)),
            spec_w((_H, _HH)),
            spec_w((_H, _HH)),
            spec_w((1, _HH)),
            spec_w((1, _HH)),
            spec_w((_HH, 128)),
            spec_w((1, 1)),
        ],
        out_specs=pl.BlockSpec((1, _NPS, 8), lambda b: (b, 0, 0)),
        out_shape=jax.ShapeDtypeStruct((_B, _NPS, 8), jnp.float32),
    )(emb_flat, doc, pos, wp, bp, w1a, w1b, w1c, b1, w2p, b2)


def _topk_body(srow_ref, scol_ref, sal_ref, psal_ref, sc_ref, w_ref, idx_ref,
               salc_ref):
    b = pl.program_id(0)
    srow = srow_ref[0]
    scol = scol_ref[0]

    smin = jnp.min(srow)
    smax = jnp.max(srow)
    spread = smax - smin
    is_norm = spread > _MIN_SIG
    denom = 1.0 / jnp.clip(spread, 1e-9, None)
    sal_row = jnp.where(is_norm, (srow - smin) * denom,
                        jnp.full_like(srow, 1.0 / _NPS))
    sal_col = jnp.where(is_norm, (scol - smin) * denom,
                        jnp.full_like(scol, 1.0 / _NPS))
    sal_ref[...] = sal_row.reshape(1, 1, _NPS)
    salc_ref[...] = sal_col

    psal = jnp.maximum(
        jnp.max(sal_col.reshape(_N * _P, _S, 1), axis=1), 0.0)
    psal_ref[...] = psal.reshape(1, _N * _P, 1)

    _CH = 16
    jj_ch = lax.broadcasted_iota(jnp.int32, (_CH, _NPS), 1)
    ii_ch = lax.broadcasted_iota(jnp.int32, (_CH, _NPS), 0)

    def _count_body(c, colsum):
        s16 = salc_ref[pl.ds(c * _CH, _CH), :]
        ii16 = c * _CH + ii_ch
        beats = (sal_row > s16) | ((sal_row == s16) & (jj_ch < ii16))
        return colsum + jnp.sum(beats.astype(jnp.float32), axis=0,
                                keepdims=True)

    colsum = lax.fori_loop(0, _NPS // _CH, _count_body,
                           jnp.zeros((1, _NPS), jnp.float32), unroll=2)
    rank_row = (_NPS - 1.0) - colsum

    jj_row = lax.broadcasted_iota(jnp.int32, (8, _NPS), 1)
    ii_base = lax.broadcasted_iota(jnp.int32, (8, _NPS), 0)
    jj_f = jj_row.astype(jnp.float32)
    sc_parts = []
    ix_parts = []
    for c in range(_KPAD // 8):
        kk8 = (c * 8 + ii_base).astype(jnp.float32)
        oh = rank_row == kk8
        sc_parts.append(jnp.sum(jnp.where(oh, sal_row, 0.0), axis=1,
                                keepdims=True))
        ix_parts.append(jnp.sum(jnp.where(oh, jj_f, 0.0), axis=1,
                                keepdims=True))
    topk_sc = jnp.concatenate(sc_parts, axis=0)
    topk_if = jnp.concatenate(ix_parts, axis=0)
    sc_ref[...] = topk_sc.reshape(1, _KPAD, 1)
    idx_ref[...] = (topk_if.astype(jnp.int32) + b * _NPS).reshape(1, _KPAD, 1)

    kmask = lax.broadcasted_iota(jnp.int32, (_KPAD, 1), 0) < _TOPK
    wmax = jnp.max(jnp.where(kmask, topk_sc, -jnp.inf), axis=0, keepdims=True)
    wexp = jnp.where(kmask, jnp.exp(topk_sc - wmax), 0.0)
    w = wexp / jnp.sum(wexp, axis=0, keepdims=True)
    w_ref[...] = w.reshape(1, _KPAD, 1)


def _topk_call(scores):
    s_row = scores.reshape(_B, 1, _NPS)
    return pl.pallas_call(
        _topk_body,
        grid=(_B,),
        in_specs=[
            pl.BlockSpec((1, 1, _NPS), lambda b: (b, 0, 0)),
            pl.BlockSpec((1, _NPS, 1), lambda b: (b, 0, 0)),
        ],
        out_specs=[
            pl.BlockSpec((1, 1, _NPS), lambda b: (b, 0, 0)),
            pl.BlockSpec((1, _N * _P, 1), lambda b: (b, 0, 0)),
            pl.BlockSpec((1, _KPAD, 1), lambda b: (b, 0, 0)),
            pl.BlockSpec((1, _KPAD, 1), lambda b: (b, 0, 0)),
            pl.BlockSpec((1, _KPAD, 1), lambda b: (b, 0, 0)),
        ],
        out_shape=[
            jax.ShapeDtypeStruct((_B, 1, _NPS), jnp.float32),
            jax.ShapeDtypeStruct((_B, _N * _P, 1), jnp.float32),
            jax.ShapeDtypeStruct((_B, _KPAD, 1), jnp.float32),
            jax.ShapeDtypeStruct((_B, _KPAD, 1), jnp.float32),
            jax.ShapeDtypeStruct((_B, _KPAD, 1), jnp.int32),
        ],
        scratch_shapes=[pltpu.VMEM((_NPS, 1), jnp.float32)],
    )(s_row, scores)


def _gather_topk(table, idx_pad):
    mesh = plsc.VectorSubcoreMesh(core_axis_name="c", subcore_axis_name="s")

    @functools.partial(
        pl.kernel, mesh=mesh,
        out_type=jax.ShapeDtypeStruct((_PAD_ROWS, _H), jnp.float32),
        scratch_types=[
            pltpu.VMEM((_ROWS_PER_W,), jnp.int32),
            pltpu.VMEM((_ROWS_PER_W, _H), jnp.float32),
            pltpu.SemaphoreType.DMA,
        ],
    )
    def k(table_hbm, idx_hbm, out_hbm, idx_v, rows_v, sem):
        wid = lax.axis_index("s") * _NC + lax.axis_index("c")
        base = wid * _ROWS_PER_W
        pltpu.sync_copy(idx_hbm.at[pl.ds(base, _ROWS_PER_W)], idx_v)
        pltpu.async_copy(table_hbm.at[idx_v], rows_v, sem).wait()
        pltpu.sync_copy(rows_v, out_hbm.at[pl.ds(base, _ROWS_PER_W)])

    return k(table, idx_pad)


def kernel(sentence_embs, paragraph_embs, document_embs, sent_valid_mask,
           para_valid_mask, Wp, bp, pos_emb, W1, b1, W2, b2):
    emb_flat = sentence_embs.reshape(_B * _NPS, _H)
    w1a = W1[:_H, :]
    w1b = W1[_H:2 * _H, :]
    w1c = W1[2 * _H:, :]
    w2p = jnp.zeros((_HH, 128), jnp.float32).at[:, 0:1].set(W2)

    scores8 = _scores_call(emb_flat, document_embs, pos_emb, Wp,
                           bp.reshape(1, _H), w1a, w1b, w1c,
                           b1.reshape(1, _HH), w2p, b2.reshape(1, 1))
    scores = scores8[:, :, 0:1]

    sal, psal, topk_sc, topk_w, topk_idx = _topk_call(scores)

    idx_flat = topk_idx.reshape(_B, _KPAD)[:, :_TOPK].reshape(_B * _TOPK)
    idx_pad = jnp.zeros((_PAD_ROWS,), jnp.int32).at[:_B * _TOPK].set(idx_flat)
    gathered = _gather_topk(emb_flat, idx_pad)

    return (sal.reshape(_B, _N, _P, _S),
            psal.reshape(_B, _N, _P),
            gathered[:_B * _TOPK].reshape(_B, _TOPK, _H),
            topk_w.reshape(_B, _KPAD)[:, :_TOPK],
            topk_sc.reshape(_B, _KPAD)[:, :_TOPK])


# speedup vs baseline: 1.6797x; 1.0652x over previous
import functools

import jax
import jax.numpy as jnp
from jax import lax
from jax.experimental import pallas as pl
from jax.experimental.pallas import tpu as pltpu
from jax.experimental.pallas import tpu_sc as plsc

_H = 768
_TOPK = 50
_KPAD = 56
_MIN_SIG = 0.05
_B, _N, _P, _S = 4, 8, 8, 16
_NPS = _N * _P * _S
_HH = _H // 2

_PREC = lax.Precision.DEFAULT

_NC, _NS = 2, 16
_NW = _NC * _NS
_ROWS_PER_W = 8
_PAD_ROWS = _NW * _ROWS_PER_W


def _bf(x):
    return x.astype(jnp.bfloat16).astype(jnp.float32)


def _dot(a, b):
    return lax.dot_general(a.astype(jnp.bfloat16), b.astype(jnp.bfloat16),
                           (((1,), (0,)), ((), ())), precision=_PREC,
                           preferred_element_type=jnp.float32)


def _score_body(emb_ref, doc_ref, pos_ref, wp_ref, bp_ref, w1a_ref, w1b_ref,
                w1c_ref, b1_ref, w2_ref, b2_ref, out_ref):
    emb = emb_ref[...]
    doc = doc_ref[0]
    pos = pos_ref[...]

    pooled = (jnp.sum(doc, axis=0, keepdims=True) - doc) * (1.0 / (_N - 1))
    base = jnp.tanh(_dot(pooled, wp_ref[...]) + bp_ref[...])
    pred = (base.reshape(_N, 1, _H) + pos.reshape(1, _S, _H)).reshape(
        _N * _S, _H)
    pnorm = jnp.clip(jnp.sqrt(jnp.sum(pred * pred, axis=1, keepdims=True)),
                     1e-12, None)
    pn = pred / pnorm
    pn_w1 = _dot(pn, w1b_ref[...])

    anorm = jnp.clip(jnp.sqrt(jnp.sum(emb * emb, axis=1, keepdims=True)),
                     1e-12, None)
    an = emb / anorm
    a_w1 = _dot(an, w1a_ref[...])

    pnb = jnp.broadcast_to(pn.reshape(_N, 1, _S, _H),
                           (_N, _P, _S, _H)).reshape(_NPS, _H)
    cos = jnp.sum(an * pnb, axis=1, keepdims=True)
    cdis = jnp.clip(1.0 - cos, 0.0, 2.0) * 0.5

    pn_w1b = jnp.broadcast_to(pn_w1.reshape(_N, 1, _S, _HH),
                              (_N, _P, _S, _HH)).reshape(_NPS, _HH)
    h_pre = a_w1 + pn_w1b + _bf(cdis) * _bf(w1c_ref[...]) + b1_ref[...]
    h = 0.5 * h_pre * (1.0 + lax.erf(h_pre * (2.0 ** -0.5)))
    d0 = _dot(h, w2_ref[...])[:, 0:1]
    refined = jax.nn.sigmoid(d0 + b2_ref[...])
    score = 0.5 * cdis + 0.5 * refined
    out_ref[...] = jnp.concatenate(
        [score, cos, cdis, a_w1[:, 0:1], pn_w1b[:, 0:1], h_pre[:, 0:1],
         h[:, 0:1], d0], axis=1).reshape(1, _NPS, 8)


def _scores_call(emb_flat, doc, pos, wp, bp, w1a, w1b, w1c, b1, w2p, b2):
    spec_w = lambda shape: pl.BlockSpec(shape, lambda b: (0,) * len(shape))
    return pl.pallas_call(
        _score_body,
        grid=(_B,),
        in_specs=[
            pl.BlockSpec((_NPS, _H), lambda b: (b, 0)),
            pl.BlockSpec((1, _N, _H), lambda b: (b, 0, 0)),
            spec_w((_S, _H)),
            spec_w((_H, _H)),
            spec_w((1, _H)),
            spec_w((_H, _HH)),
            spec_w((_H, _HH)),
            spec_w((1, _HH)),
            spec_w((1, _HH)),
            spec_w((_HH, 128)),
            spec_w((1, 1)),
        ],
        out_specs=pl.BlockSpec((1, _NPS, 8), lambda b: (b, 0, 0)),
        out_shape=jax.ShapeDtypeStruct((_B, _NPS, 8), jnp.float32),
    )(emb_flat, doc, pos, wp, bp, w1a, w1b, w1c, b1, w2p, b2)


def _topk_body(srow_ref, scol_ref, sal_ref, psal_ref, sc_ref, w_ref, idx_ref,
               salc_ref):
    b = pl.program_id(0)
    srow = srow_ref[0]
    scol = scol_ref[0]

    smin = jnp.min(srow)
    smax = jnp.max(srow)
    spread = smax - smin
    is_norm = spread > _MIN_SIG
    denom = 1.0 / jnp.clip(spread, 1e-9, None)
    sal_row = jnp.where(is_norm, (srow - smin) * denom,
                        jnp.full_like(srow, 1.0 / _NPS))
    sal_col = jnp.where(is_norm, (scol - smin) * denom,
                        jnp.full_like(scol, 1.0 / _NPS))
    sal_ref[...] = sal_row.reshape(1, 1, _NPS)
    salc_ref[...] = sal_col

    psal = jnp.maximum(
        jnp.max(sal_col.reshape(_N * _P, _S, 1), axis=1), 0.0)
    psal_ref[...] = psal.reshape(1, _N * _P, 1)

    _CH = 32
    jj_ch = lax.broadcasted_iota(jnp.int32, (_CH, _NPS), 1)
    ii_ch = lax.broadcasted_iota(jnp.int32, (_CH, _NPS), 0)

    def _count_body(c, colsum):
        s16 = salc_ref[pl.ds(c * _CH, _CH), :]
        ii16 = c * _CH + ii_ch
        beats = (sal_row > s16) | ((sal_row == s16) & (jj_ch < ii16))
        return colsum + jnp.sum(beats.astype(jnp.float32), axis=0,
                                keepdims=True)

    colsum = lax.fori_loop(0, _NPS // _CH, _count_body,
                           jnp.zeros((1, _NPS), jnp.float32), unroll=2)
    rank_row = (_NPS - 1.0) - colsum

    jj_row = lax.broadcasted_iota(jnp.int32, (8, _NPS), 1)
    ii_base = lax.broadcasted_iota(jnp.int32, (8, _NPS), 0)
    jj_f = jj_row.astype(jnp.float32)
    sc_parts = []
    ix_parts = []
    for c in range(_KPAD // 8):
        kk8 = (c * 8 + ii_base).astype(jnp.float32)
        oh = rank_row == kk8
        sc_parts.append(jnp.sum(jnp.where(oh, sal_row, 0.0), axis=1,
                                keepdims=True))
        ix_parts.append(jnp.sum(jnp.where(oh, jj_f, 0.0), axis=1,
                                keepdims=True))
    topk_sc = jnp.concatenate(sc_parts, axis=0)
    topk_if = jnp.concatenate(ix_parts, axis=0)
    sc_ref[...] = topk_sc.reshape(1, _KPAD, 1)
    idx_ref[...] = (topk_if.astype(jnp.int32) + b * _NPS).reshape(1, _KPAD, 1)

    kmask = lax.broadcasted_iota(jnp.int32, (_KPAD, 1), 0) < _TOPK
    wmax = jnp.max(jnp.where(kmask, topk_sc, -jnp.inf), axis=0, keepdims=True)
    wexp = jnp.where(kmask, jnp.exp(topk_sc - wmax), 0.0)
    w = wexp / jnp.sum(wexp, axis=0, keepdims=True)
    w_ref[...] = w.reshape(1, _KPAD, 1)


def _topk_call(scores):
    s_row = scores.reshape(_B, 1, _NPS)
    return pl.pallas_call(
        _topk_body,
        grid=(_B,),
        in_specs=[
            pl.BlockSpec((1, 1, _NPS), lambda b: (b, 0, 0)),
            pl.BlockSpec((1, _NPS, 1), lambda b: (b, 0, 0)),
        ],
        out_specs=[
            pl.BlockSpec((1, 1, _NPS), lambda b: (b, 0, 0)),
            pl.BlockSpec((1, _N * _P, 1), lambda b: (b, 0, 0)),
            pl.BlockSpec((1, _KPAD, 1), lambda b: (b, 0, 0)),
            pl.BlockSpec((1, _KPAD, 1), lambda b: (b, 0, 0)),
            pl.BlockSpec((1, _KPAD, 1), lambda b: (b, 0, 0)),
        ],
        out_shape=[
            jax.ShapeDtypeStruct((_B, 1, _NPS), jnp.float32),
            jax.ShapeDtypeStruct((_B, _N * _P, 1), jnp.float32),
            jax.ShapeDtypeStruct((_B, _KPAD, 1), jnp.float32),
            jax.ShapeDtypeStruct((_B, _KPAD, 1), jnp.float32),
            jax.ShapeDtypeStruct((_B, _KPAD, 1), jnp.int32),
        ],
        scratch_shapes=[pltpu.VMEM((_NPS, 1), jnp.float32)],
    )(s_row, scores)


def _gather_topk(table, idx_pad):
    mesh = plsc.VectorSubcoreMesh(core_axis_name="c", subcore_axis_name="s")

    @functools.partial(
        pl.kernel, mesh=mesh,
        out_type=jax.ShapeDtypeStruct((_PAD_ROWS, _H), jnp.float32),
        scratch_types=[
            pltpu.VMEM((_ROWS_PER_W,), jnp.int32),
            pltpu.VMEM((_ROWS_PER_W, _H), jnp.float32),
            pltpu.SemaphoreType.DMA,
        ],
    )
    def k(table_hbm, idx_hbm, out_hbm, idx_v, rows_v, sem):
        wid = lax.axis_index("s") * _NC + lax.axis_index("c")
        base = wid * _ROWS_PER_W
        pltpu.sync_copy(idx_hbm.at[pl.ds(base, _ROWS_PER_W)], idx_v)
        pltpu.async_copy(table_hbm.at[idx_v], rows_v, sem).wait()
        pltpu.sync_copy(rows_v, out_hbm.at[pl.ds(base, _ROWS_PER_W)])

    return k(table, idx_pad)


def kernel(sentence_embs, paragraph_embs, document_embs, sent_valid_mask,
           para_valid_mask, Wp, bp, pos_emb, W1, b1, W2, b2):
    emb_flat = sentence_embs.reshape(_B * _NPS, _H)
    w1a = W1[:_H, :]
    w1b = W1[_H:2 * _H, :]
    w1c = W1[2 * _H:, :]
    w2p = jnp.zeros((_HH, 128), jnp.float32).at[:, 0:1].set(W2)

    scores8 = _scores_call(emb_flat, document_embs, pos_emb, Wp,
                           bp.reshape(1, _H), w1a, w1b, w1c,
                           b1.reshape(1, _HH), w2p, b2.reshape(1, 1))
    scores = scores8[:, :, 0:1]

    sal, psal, topk_sc, topk_w, topk_idx = _topk_call(scores)

    idx_flat = topk_idx.reshape(_B, _KPAD)[:, :_TOPK].reshape(_B * _TOPK)
    idx_pad = jnp.zeros((_PAD_ROWS,), jnp.int32).at[:_B * _TOPK].set(idx_flat)
    gathered = _gather_topk(emb_flat, idx_pad)

    return (sal.reshape(_B, _N, _P, _S),
            psal.reshape(_B, _N, _P),
            gathered[:_B * _TOPK].reshape(_B, _TOPK, _H),
            topk_w.reshape(_B, _KPAD)[:, :_TOPK],
            topk_sc.reshape(_B, _KPAD)[:, :_TOPK])


# speedup vs baseline: 1.7420x; 1.0371x over previous
import functools

import jax
import jax.numpy as jnp
from jax import lax
from jax.experimental import pallas as pl
from jax.experimental.pallas import tpu as pltpu
from jax.experimental.pallas import tpu_sc as plsc

_H = 768
_TOPK = 50
_KPAD = 56
_MIN_SIG = 0.05
_B, _N, _P, _S = 4, 8, 8, 16
_NPS = _N * _P * _S
_HH = _H // 2

_PREC = lax.Precision.DEFAULT

_NC, _NS = 2, 16
_NW = _NC * _NS
_ROWS_PER_W = 8
_PAD_ROWS = _NW * _ROWS_PER_W


def _bf(x):
    return x.astype(jnp.bfloat16).astype(jnp.float32)


def _dot(a, b):
    return lax.dot_general(a.astype(jnp.bfloat16), b.astype(jnp.bfloat16),
                           (((1,), (0,)), ((), ())), precision=_PREC,
                           preferred_element_type=jnp.float32)


def _score_body(emb_ref, doc_ref, pos_ref, wp_ref, bp_ref, w1a_ref, w1b_ref,
                w1c_ref, b1_ref, w2_ref, b2_ref, out_ref):
    emb = emb_ref[...]
    doc = doc_ref[0]
    pos = pos_ref[...]

    pooled = (jnp.sum(doc, axis=0, keepdims=True) - doc) * (1.0 / (_N - 1))
    base = jnp.tanh(_dot(pooled, wp_ref[...]) + bp_ref[...])
    pred = (base.reshape(_N, 1, _H) + pos.reshape(1, _S, _H)).reshape(
        _N * _S, _H)
    pnorm = jnp.clip(jnp.sqrt(jnp.sum(pred * pred, axis=1, keepdims=True)),
                     1e-12, None)
    pn = pred / pnorm
    pn_w1 = _dot(pn, w1b_ref[...])

    anorm = jnp.clip(jnp.sqrt(jnp.sum(emb * emb, axis=1, keepdims=True)),
                     1e-12, None)
    an = emb / anorm
    a_w1 = _dot(an, w1a_ref[...])

    pnb = jnp.broadcast_to(pn.reshape(_N, 1, _S, _H),
                           (_N, _P, _S, _H)).reshape(_NPS, _H)
    cos = jnp.sum(an * pnb, axis=1, keepdims=True)
    cdis = jnp.clip(1.0 - cos, 0.0, 2.0) * 0.5

    pn_w1b = jnp.broadcast_to(pn_w1.reshape(_N, 1, _S, _HH),
                              (_N, _P, _S, _HH)).reshape(_NPS, _HH)
    h_pre = a_w1 + pn_w1b + _bf(cdis) * _bf(w1c_ref[...]) + b1_ref[...]
    h = 0.5 * h_pre * (1.0 + lax.erf(h_pre * (2.0 ** -0.5)))
    d0 = _dot(h, w2_ref[...])[:, 0:1]
    refined = jax.nn.sigmoid(d0 + b2_ref[...])
    score = 0.5 * cdis + 0.5 * refined
    out_ref[...] = jnp.concatenate(
        [score, cos, cdis, a_w1[:, 0:1], pn_w1b[:, 0:1], h_pre[:, 0:1],
         h[:, 0:1], d0], axis=1).reshape(1, _NPS, 8)


def _scores_call(emb_flat, doc, pos, wp, bp, w1a, w1b, w1c, b1, w2p, b2):
    spec_w = lambda shape: pl.BlockSpec(shape, lambda b: (0,) * len(shape))
    return pl.pallas_call(
        _score_body,
        grid=(_B,),
        in_specs=[
            pl.BlockSpec((_NPS, _H), lambda b: (b, 0)),
            pl.BlockSpec((1, _N, _H), lambda b: (b, 0, 0)),
            spec_w((_S, _H)),
            spec_w((_H, _H)),
            spec_w((1, _H)),
            spec_w((_H, _HH)),
            spec_w((_H, _HH)),
            spec_w((1, _HH)),
            spec_w((1, _HH)),
            spec_w((_HH, 128)),
            spec_w((1, 1)),
        ],
        out_specs=pl.BlockSpec((1, _NPS, 8), lambda b: (b, 0, 0)),
        out_shape=jax.ShapeDtypeStruct((_B, _NPS, 8), jnp.float32),
    )(emb_flat, doc, pos, wp, bp, w1a, w1b, w1c, b1, w2p, b2)


def _topk_body(srow_ref, scol_ref, sal_ref, psal_ref, sc_ref, w_ref, idx_ref,
               salc_ref):
    b = pl.program_id(0)
    srow = srow_ref[0]
    scol = scol_ref[0]

    smin = jnp.min(srow)
    smax = jnp.max(srow)
    spread = smax - smin
    is_norm = spread > _MIN_SIG
    denom = 1.0 / jnp.clip(spread, 1e-9, None)
    sal_row = jnp.where(is_norm, (srow - smin) * denom,
                        jnp.full_like(srow, 1.0 / _NPS))
    sal_col = jnp.where(is_norm, (scol - smin) * denom,
                        jnp.full_like(scol, 1.0 / _NPS))
    sal_ref[...] = sal_row.reshape(1, 1, _NPS)
    salc_ref[...] = sal_col

    psal = jnp.maximum(
        jnp.max(sal_col.reshape(_N * _P, _S, 1), axis=1), 0.0)
    psal_ref[...] = psal.reshape(1, _N * _P, 1)

    _CH = 64
    jj_ch = lax.broadcasted_iota(jnp.int32, (_CH, _NPS), 1)
    ii_ch = lax.broadcasted_iota(jnp.int32, (_CH, _NPS), 0)

    def _count_body(c, colsum):
        s16 = salc_ref[pl.ds(c * _CH, _CH), :]
        ii16 = c * _CH + ii_ch
        beats = (sal_row > s16) | ((sal_row == s16) & (jj_ch < ii16))
        return colsum + jnp.sum(beats.astype(jnp.float32), axis=0,
                                keepdims=True)

    colsum = lax.fori_loop(0, _NPS // _CH, _count_body,
                           jnp.zeros((1, _NPS), jnp.float32), unroll=2)
    rank_row = (_NPS - 1.0) - colsum

    jj_row = lax.broadcasted_iota(jnp.int32, (8, _NPS), 1)
    ii_base = lax.broadcasted_iota(jnp.int32, (8, _NPS), 0)
    jj_f = jj_row.astype(jnp.float32)
    sc_parts = []
    ix_parts = []
    for c in range(_KPAD // 8):
        kk8 = (c * 8 + ii_base).astype(jnp.float32)
        oh = rank_row == kk8
        sc_parts.append(jnp.sum(jnp.where(oh, sal_row, 0.0), axis=1,
                                keepdims=True))
        ix_parts.append(jnp.sum(jnp.where(oh, jj_f, 0.0), axis=1,
                                keepdims=True))
    topk_sc = jnp.concatenate(sc_parts, axis=0)
    topk_if = jnp.concatenate(ix_parts, axis=0)
    sc_ref[...] = topk_sc.reshape(1, _KPAD, 1)
    idx_ref[...] = (topk_if.astype(jnp.int32) + b * _NPS).reshape(1, _KPAD, 1)

    kmask = lax.broadcasted_iota(jnp.int32, (_KPAD, 1), 0) < _TOPK
    wmax = jnp.max(jnp.where(kmask, topk_sc, -jnp.inf), axis=0, keepdims=True)
    wexp = jnp.where(kmask, jnp.exp(topk_sc - wmax), 0.0)
    w = wexp / jnp.sum(wexp, axis=0, keepdims=True)
    w_ref[...] = w.reshape(1, _KPAD, 1)


def _topk_call(scores):
    s_row = scores.reshape(_B, 1, _NPS)
    return pl.pallas_call(
        _topk_body,
        grid=(_B,),
        in_specs=[
            pl.BlockSpec((1, 1, _NPS), lambda b: (b, 0, 0)),
            pl.BlockSpec((1, _NPS, 1), lambda b: (b, 0, 0)),
        ],
        out_specs=[
            pl.BlockSpec((1, 1, _NPS), lambda b: (b, 0, 0)),
            pl.BlockSpec((1, _N * _P, 1), lambda b: (b, 0, 0)),
            pl.BlockSpec((1, _KPAD, 1), lambda b: (b, 0, 0)),
            pl.BlockSpec((1, _KPAD, 1), lambda b: (b, 0, 0)),
            pl.BlockSpec((1, _KPAD, 1), lambda b: (b, 0, 0)),
        ],
        out_shape=[
            jax.ShapeDtypeStruct((_B, 1, _NPS), jnp.float32),
            jax.ShapeDtypeStruct((_B, _N * _P, 1), jnp.float32),
            jax.ShapeDtypeStruct((_B, _KPAD, 1), jnp.float32),
            jax.ShapeDtypeStruct((_B, _KPAD, 1), jnp.float32),
            jax.ShapeDtypeStruct((_B, _KPAD, 1), jnp.int32),
        ],
        scratch_shapes=[pltpu.VMEM((_NPS, 1), jnp.float32)],
    )(s_row, scores)


def _gather_topk(table, idx_pad):
    mesh = plsc.VectorSubcoreMesh(core_axis_name="c", subcore_axis_name="s")

    @functools.partial(
        pl.kernel, mesh=mesh,
        out_type=jax.ShapeDtypeStruct((_PAD_ROWS, _H), jnp.float32),
        scratch_types=[
            pltpu.VMEM((_ROWS_PER_W,), jnp.int32),
            pltpu.VMEM((_ROWS_PER_W, _H), jnp.float32),
            pltpu.SemaphoreType.DMA,
        ],
    )
    def k(table_hbm, idx_hbm, out_hbm, idx_v, rows_v, sem):
        wid = lax.axis_index("s") * _NC + lax.axis_index("c")
        base = wid * _ROWS_PER_W
        pltpu.sync_copy(idx_hbm.at[pl.ds(base, _ROWS_PER_W)], idx_v)
        pltpu.async_copy(table_hbm.at[idx_v], rows_v, sem).wait()
        pltpu.sync_copy(rows_v, out_hbm.at[pl.ds(base, _ROWS_PER_W)])

    return k(table, idx_pad)


def kernel(sentence_embs, paragraph_embs, document_embs, sent_valid_mask,
           para_valid_mask, Wp, bp, pos_emb, W1, b1, W2, b2):
    emb_flat = sentence_embs.reshape(_B * _NPS, _H)
    w1a = W1[:_H, :]
    w1b = W1[_H:2 * _H, :]
    w1c = W1[2 * _H:, :]
    w2p = jnp.zeros((_HH, 128), jnp.float32).at[:, 0:1].set(W2)

    scores8 = _scores_call(emb_flat, document_embs, pos_emb, Wp,
                           bp.reshape(1, _H), w1a, w1b, w1c,
                           b1.reshape(1, _HH), w2p, b2.reshape(1, 1))
    scores = scores8[:, :, 0:1]

    sal, psal, topk_sc, topk_w, topk_idx = _topk_call(scores)

    idx_flat = topk_idx.reshape(_B, _KPAD)[:, :_TOPK].reshape(_B * _TOPK)
    idx_pad = jnp.zeros((_PAD_ROWS,), jnp.int32).at[:_B * _TOPK].set(idx_flat)
    gathered = _gather_topk(emb_flat, idx_pad)

    return (sal.reshape(_B, _N, _P, _S),
            psal.reshape(_B, _N, _P),
            gathered[:_B * _TOPK].reshape(_B, _TOPK, _H),
            topk_w.reshape(_B, _KPAD)[:, :_TOPK],
            topk_sc.reshape(_B, _KPAD)[:, :_TOPK])
